# baseline (speedup 1.0000x reference)
import functools

import jax
import jax.numpy as jnp
from jax import lax
from jax.experimental import pallas as pl
from jax.experimental.pallas import tpu as pltpu
from jax.experimental.pallas import tpu_sc as plsc

VOCAB = 1000000
EMBED_DIM = 16
BATCH = 16384
FIELDS = 26

NC = 2
NS = 16
NW = NC * NS

ROWS_PER_W = BATCH // NW
CHUNK = 128
NCHUNK = ROWS_PER_W // CHUNK
IDX_PER_CHUNK = CHUNK * FIELDS
NDMA = IDX_PER_CHUNK // 128


def _fm_kernel(idx_hbm, emb_hbm, lin_hbm, bias_hbm, out_hbm,
               idx_v, emb_v, lin_v, out_v, bias_v, sem):
    wid = lax.axis_index("s") * NC + lax.axis_index("c")
    lin_flat_hbm = lin_hbm

    pltpu.sync_copy(bias_hbm, bias_v.at[pl.ds(0, 1)])
    pltpu.sync_copy(idx_hbm.at[wid], idx_v)

    lane = lax.iota(jnp.int32, 16)
    tail_mask = lane < (FIELDS - 16)
    bias_vec = jnp.where(lane == 0, bias_v[pl.ds(0, 16)], 0.0)

    for g in range(NCHUNK):
        base = wid * ROWS_PER_W + g * CHUNK

        def fire(j, carry):
            pltpu.make_async_copy(
                emb_hbm.at[idx_v.at[g * NDMA + j]],
                emb_v.at[pl.ds(j * 128, 128)], sem).start()
            pltpu.make_async_copy(
                lin_flat_hbm.at[idx_v.at[g * NDMA + j]],
                lin_v.at[pl.ds(j * 128, 128)], sem).start()
            return carry
        lax.fori_loop(0, NDMA, fire, 0)

        def drain(j, carry):
            pltpu.make_async_copy(
                emb_hbm.at[idx_v.at[g * NDMA + j]],
                emb_v.at[pl.ds(j * 128, 128)], sem).wait()
            pltpu.make_async_copy(
                lin_flat_hbm.at[idx_v.at[g * NDMA + j]],
                lin_v.at[pl.ds(j * 128, 128)], sem).wait()
            return carry
        lax.fori_loop(0, NDMA, drain, 0)

        def group_body(gr, carry):
            def row_body(k, acc):
                r = gr * 16 + k
                rbase = r * FIELDS
                s = emb_v[rbase, :]
                ssq = s * s
                for f in range(1, FIELDS):
                    v = emb_v[rbase + f, :]
                    s = s + v
                    ssq = ssq + v * v
                l0 = lin_v[pl.ds(rbase, 16)]
                l1 = lin_v[pl.ds(rbase + 16, 16)]
                u = (0.5 * (s * s - ssq) + l0
                     + jnp.where(tail_mask, l1, 0.0) + bias_vec)
                tot = jnp.sum(u)
                return jnp.where(lane == k, tot, acc)
            acc = lax.fori_loop(0, 16, row_body, jnp.zeros((16,), jnp.float32))
            out_v[pl.ds(gr * 16, 16)] = acc
            return carry
        lax.fori_loop(0, CHUNK // 16, group_body, 0)

        pltpu.sync_copy(out_v, out_hbm.at[pl.ds(base, CHUNK)])


_RL_C = 2000
_RL_NCHUNK = VOCAB // _RL_C
_RL_KMAX = (_RL_NCHUNK + NW - 1) // NW


def _relayout_body(embt_hbm, out_hbm, in_v, out_v):
    wid = lax.axis_index("s") * NC + lax.axis_index("c")
    lane = lax.iota(jnp.int32, 16)

    def step(k, carry):
        c = k * NW + wid

        @pl.when(c < _RL_NCHUNK)
        def _():
            pltpu.sync_copy(embt_hbm.at[:, pl.ds(c * _RL_C, _RL_C)], in_v)

            def col(u, cc):
                for s in range(8):
                    j = u * 8 + s
                    v = plsc.load_gather(in_v, [lane, jnp.full((16,), j, jnp.int32)])
                    out_v[j, :] = v
                return cc
            lax.fori_loop(0, _RL_C // 8, col, 0)
            pltpu.sync_copy(out_v, out_hbm.at[pl.ds(c * _RL_C, _RL_C)])
        return carry
    lax.fori_loop(0, _RL_KMAX, step, 0)


def _relayout_emb(emb_t):
    mesh = plsc.VectorSubcoreMesh(core_axis_name="c", subcore_axis_name="s")
    run = pl.kernel(
        _relayout_body,
        mesh=mesh,
        compiler_params=pltpu.CompilerParams(
            needs_layout_passes=False, use_tc_tiling_on_sc=False),
        out_type=jax.ShapeDtypeStruct((VOCAB, EMBED_DIM), jnp.float32),
        scratch_types=[
            pltpu.VMEM((EMBED_DIM, _RL_C), jnp.float32),
            pltpu.VMEM((_RL_C, EMBED_DIM), jnp.float32),
        ],
    )
    return run(emb_t)


@jax.jit
def kernel(interaction_pairs, emb_table, lin_table, bias):
    idx3d = interaction_pairs.astype(jnp.int32).reshape(NW, NCHUNK * NDMA, 128)
    lin_flat = lin_table.reshape(-1)
    emb_rm = _relayout_emb(emb_table.T)

    mesh = plsc.VectorSubcoreMesh(core_axis_name="c", subcore_axis_name="s")
    run = pl.kernel(
        _fm_kernel,
        mesh=mesh,
        compiler_params=pltpu.CompilerParams(
            needs_layout_passes=False, use_tc_tiling_on_sc=False),
        out_type=jax.ShapeDtypeStruct((BATCH,), jnp.float32),
        scratch_types=[
            pltpu.VMEM((NCHUNK * NDMA, 128), jnp.int32),
            pltpu.VMEM((IDX_PER_CHUNK, 16), jnp.float32),
            pltpu.VMEM((IDX_PER_CHUNK + 16,), jnp.float32),
            pltpu.VMEM((CHUNK,), jnp.float32),
            pltpu.VMEM((16,), jnp.float32),
            pltpu.SemaphoreType.DMA,
        ],
    )
    return run(idx3d, emb_rm, lin_flat, bias)


# speedup vs baseline: 3.2665x; 3.2665x over previous
import jax
import jax.numpy as jnp
from jax import lax
from jax.experimental import pallas as pl
from jax.experimental.pallas import tpu as pltpu
from jax.experimental.pallas import tpu_sc as plsc

VOCAB = 1000000
EMBED_DIM = 16
BATCH = 16384
FIELDS = 26

NC = 2
NS = 16
NW = NC * NS

ROWS_PER_W = BATCH // NW
CHUNK = 128
NCHUNK = ROWS_PER_W // CHUNK
IDX_PER_CHUNK = CHUNK * FIELDS
NDMA = IDX_PER_CHUNK // 128


def _fm_kernel(idx_hbm, emb_hbm, lin_hbm, bias_hbm, out_hbm,
               idx_v, emb_v, lin_v, out_v, bias_v, sem0, sem1):
    wid = lax.axis_index("s") * NC + lax.axis_index("c")

    pltpu.sync_copy(bias_hbm, bias_v.at[pl.ds(0, 1)])
    pltpu.sync_copy(idx_hbm.at[wid], idx_v)

    lane = lax.iota(jnp.int32, 16)
    tail_mask = lane < (FIELDS - 16)
    bias_vec = jnp.where(lane == 0, bias_v[pl.ds(0, 16)], 0.0)

    sems = (sem0, sem1)

    def chunk_fire(g, b):
        def fj(j, carry):
            pltpu.make_async_copy(
                emb_hbm.at[idx_v.at[g * NDMA + j]],
                emb_v.at[b, pl.ds(j * 128, 128)], sems[b]).start()
            pltpu.make_async_copy(
                lin_hbm.at[idx_v.at[g * NDMA + j]],
                lin_v.at[b, pl.ds(j * 128, 128)], sems[b]).start()
            return carry
        lax.fori_loop(0, NDMA, fj, 0)

    def chunk_drain(g, b):
        def dj(j, carry):
            pltpu.make_async_copy(
                emb_hbm.at[idx_v.at[g * NDMA + j]],
                emb_v.at[b, pl.ds(j * 128, 128)], sems[b]).wait()
            pltpu.make_async_copy(
                lin_hbm.at[idx_v.at[g * NDMA + j]],
                lin_v.at[b, pl.ds(j * 128, 128)], sems[b]).wait()
            return carry
        lax.fori_loop(0, NDMA, dj, 0)

    def chunk_compute(g, b):
        def group_body(gr, carry):
            def row_body(k, acc):
                r = gr * 16 + k
                rbase = r * FIELDS
                s = emb_v[b, rbase, :]
                ssq = s * s
                for f in range(1, FIELDS):
                    v = emb_v[b, rbase + f, :]
                    s = s + v
                    ssq = ssq + v * v
                l0 = lin_v[b, pl.ds(rbase, 16)]
                l1 = lin_v[b, pl.ds(rbase + 16, 16)]
                u = (0.5 * (s * s - ssq) + l0
                     + jnp.where(tail_mask, l1, 0.0) + bias_vec)
                tot = jnp.sum(u)
                return jnp.where(lane == k, tot, acc)
            acc = lax.fori_loop(0, 16, row_body, jnp.zeros((16,), jnp.float32))
            out_v[pl.ds(gr * 16, 16)] = acc
            return carry
        lax.fori_loop(0, CHUNK // 16, group_body, 0)

    chunk_fire(0, 0)
    for g in range(NCHUNK):
        b = g & 1
        if g + 1 < NCHUNK:
            chunk_fire(g + 1, (g + 1) & 1)
        chunk_drain(g, b)
        chunk_compute(g, b)
        pltpu.sync_copy(
            out_v, out_hbm.at[pl.ds(wid * ROWS_PER_W + g * CHUNK, CHUNK)])


@jax.jit
def kernel(interaction_pairs, emb_table, lin_table, bias):
    idx3d = interaction_pairs.astype(jnp.int32).reshape(NW, NCHUNK * NDMA, 128)
    lin_flat = lin_table.reshape(-1)

    mesh = plsc.VectorSubcoreMesh(core_axis_name="c", subcore_axis_name="s")
    run = pl.kernel(
        _fm_kernel,
        mesh=mesh,
        compiler_params=pltpu.CompilerParams(
            needs_layout_passes=False, use_tc_tiling_on_sc=False),
        out_type=jax.ShapeDtypeStruct((BATCH,), jnp.float32),
        scratch_types=[
            pltpu.VMEM((NCHUNK * NDMA, 128), jnp.int32),
            pltpu.VMEM((2, IDX_PER_CHUNK, 16), jnp.float32),
            pltpu.VMEM((2, IDX_PER_CHUNK + 16), jnp.float32),
            pltpu.VMEM((CHUNK,), jnp.float32),
            pltpu.VMEM((16,), jnp.float32),
            pltpu.SemaphoreType.DMA,
            pltpu.SemaphoreType.DMA,
        ],
    )
    return run(idx3d, emb_table, lin_flat, bias)
